# Initial kernel scaffold; baseline (speedup 1.0000x reference)
import jax
import jax.numpy as jnp
from jax.experimental import pallas as pl


def kernel(H, X_node):
    raise NotImplementedError("write your pallas kernel here")


# speedup vs baseline: 4.3138x; 4.3138x over previous
import functools

import jax
import jax.numpy as jnp
from jax import lax
from jax.experimental import pallas as pl
from jax.experimental.pallas import tpu as pltpu
from jax.experimental.pallas import tpu_sc as plsc

V = 10000
E = 320000
D = 128

NC = 2
NS = 16
NW = NC * NS
BLK = 128
NB = E // BLK
V_TILE = 624
V_LAST = V - (NS - 1) * V_TILE


def _acc_slab(s):
    return pl.ds(s * V_TILE, V_TILE)


def _sc_body(h_hbm, ids_hbm, zeros_hbm, part_hbm, idx_v, rows_v, acc):
    c = lax.axis_index("c")
    s = lax.axis_index("s")
    wid = c * NS + s

    pltpu.sync_copy(zeros_hbm.at[_acc_slab(s)], acc.at[_acc_slab(s)])

    @pl.when(s == NS - 1)
    def _():
        tail = pl.ds((NS - 1) * V_TILE + V_TILE, V_LAST - V_TILE)
        pltpu.sync_copy(zeros_hbm.at[tail], acc.at[tail])

    plsc.subcore_barrier()

    base = NB // NW
    extra = NB - base * NW
    start = wid * base + jnp.minimum(wid, extra)
    end = start + base + jnp.where(wid < extra, 1, 0)

    def step(b, carry):
        pltpu.sync_copy(ids_hbm.at[pl.ds(b * BLK, BLK)], idx_v)
        pltpu.sync_copy(h_hbm.at[pl.ds(b * BLK, BLK)], rows_v)
        pltpu.sync_copy(rows_v, acc.at[idx_v], add=True)
        return carry

    lax.fori_loop(start, end, step, 0)
    plsc.subcore_barrier()

    pltpu.sync_copy(acc.at[_acc_slab(s)], part_hbm.at[c, _acc_slab(s)])

    @pl.when(s == NS - 1)
    def _():
        tail = pl.ds((NS - 1) * V_TILE + V_TILE, V_LAST - V_TILE)
        pltpu.sync_copy(acc.at[tail], part_hbm.at[c, tail])


@jax.jit
def _segment_sum_sc(H, ids, zeros):
    mesh = plsc.VectorSubcoreMesh(core_axis_name="c", subcore_axis_name="s")
    return pl.kernel(
        _sc_body,
        out_type=jax.ShapeDtypeStruct((NC, V, D), jnp.float32),
        mesh=mesh,
        scratch_types=[
            pltpu.VMEM((BLK,), jnp.int32),
            pltpu.VMEM((BLK, D), jnp.float32),
            pltpu.VMEM_SHARED((V, D), jnp.float32),
        ],
    )(H, ids, zeros)


def _tc_add_body(p0, p1, o):
    o[...] = p0[...] + p1[...]


@jax.jit
def _combine(part):
    blk = 1000
    grid = V // blk
    spec = pl.BlockSpec((blk, D), lambda i: (i, 0))
    return pl.pallas_call(
        _tc_add_body,
        grid=(grid,),
        in_specs=[spec, spec],
        out_specs=spec,
        out_shape=jax.ShapeDtypeStruct((V, D), jnp.float32),
    )(part[0], part[1])


def kernel(H, X_node):
    ids = X_node.astype(jnp.int32)
    zeros = jnp.zeros((V, D), jnp.float32)
    part = _segment_sum_sc(H, ids, zeros)
    return _combine(part)


# speedup vs baseline: 7.1058x; 1.6472x over previous
import jax
import jax.numpy as jnp
from jax import lax
from jax.experimental import pallas as pl
from jax.experimental.pallas import tpu as pltpu
from jax.experimental.pallas import tpu_sc as plsc

V = 10000
E = 320000
D = 128

NC = 2
NS = 16
NW = NC * NS
BLK = 128
NB = E // BLK
CHB = 1
CH = CHB * BLK
WBLK = 80
LBLK = NB - (NW - 1) * WBLK
V_TILE = 624
V_LAST = V - (NS - 1) * V_TILE


def _acc_slab(s):
    return pl.ds(s * V_TILE, V_TILE)


def _acc_tail():
    return pl.ds((NS - 1) * V_TILE + V_TILE, V_LAST - V_TILE)


def _sc_body(h_hbm, ids_hbm, zeros_hbm, part_hbm,
             idx_all, rows0, rows1, sem0, sem1, acc):
    c = lax.axis_index("c")
    s = lax.axis_index("s")
    wid = c * NS + s

    @pl.when(wid < NW - 1)
    def _():
        pltpu.sync_copy(ids_hbm.at[pl.ds(wid * WBLK, WBLK)], idx_all)

    @pl.when(wid == NW - 1)
    def _():
        pltpu.sync_copy(ids_hbm.at[pl.ds((NW - 1) * WBLK, LBLK)],
                        idx_all.at[pl.ds(0, LBLK)])

    pltpu.sync_copy(zeros_hbm.at[_acc_slab(s)], acc.at[_acc_slab(s)])

    @pl.when(s == NS - 1)
    def _():
        pltpu.sync_copy(zeros_hbm.at[_acc_tail()], acc.at[_acc_tail()])

    plsc.subcore_barrier()

    b0 = jnp.where(wid < NW - 1, wid * WBLK, (NW - 1) * WBLK)
    nch = jnp.where(wid < NW - 1, WBLK // CHB, LBLK // CHB)
    bufs = ((rows0, sem0), (rows1, sem1))

    def start(ch, k):
        rows, sem = bufs[k]
        pltpu.async_copy(h_hbm.at[pl.ds((b0 + ch * CHB) * BLK, CH)], rows, sem)

    def wait(k):
        rows, sem = bufs[k]
        pltpu.make_async_copy(h_hbm.at[pl.ds(0, CH)], rows, sem).wait()

    start(0, 0)
    start(1, 1)

    def pair(p, carry):
        for k in (0, 1):
            ch = 2 * p + k
            wait(k)
            rows, _ = bufs[k]
            for j in range(CHB):
                pltpu.sync_copy(rows.at[pl.ds(j * BLK, BLK)],
                                acc.at[idx_all.at[ch * CHB + j]], add=True)

            @pl.when(ch + 2 < nch)
            def _():
                start(ch + 2, k)
        return carry

    lax.fori_loop(0, nch // 2, pair, 0)
    plsc.subcore_barrier()

    pltpu.sync_copy(acc.at[_acc_slab(s)], part_hbm.at[c, _acc_slab(s)])

    @pl.when(s == NS - 1)
    def _():
        pltpu.sync_copy(acc.at[_acc_tail()], part_hbm.at[c, _acc_tail()])


@jax.jit
def _segment_sum_sc(H, ids2, zeros):
    mesh = plsc.VectorSubcoreMesh(core_axis_name="c", subcore_axis_name="s")
    return pl.kernel(
        _sc_body,
        out_type=jax.ShapeDtypeStruct((NC, V, D), jnp.float32),
        mesh=mesh,
        scratch_types=[
            pltpu.VMEM((WBLK, BLK), jnp.int32),
            pltpu.VMEM((CH, D), jnp.float32),
            pltpu.VMEM((CH, D), jnp.float32),
            pltpu.SemaphoreType.DMA,
            pltpu.SemaphoreType.DMA,
            pltpu.VMEM_SHARED((V, D), jnp.float32),
        ],
    )(H, ids2, zeros)


def _tc_add_body(p0, p1, o):
    o[...] = p0[...] + p1[...]


@jax.jit
def _combine(part):
    blk = 1000
    grid = V // blk
    spec = pl.BlockSpec((blk, D), lambda i: (i, 0))
    return pl.pallas_call(
        _tc_add_body,
        grid=(grid,),
        in_specs=[spec, spec],
        out_specs=spec,
        out_shape=jax.ShapeDtypeStruct((V, D), jnp.float32),
    )(part[0], part[1])


def kernel(H, X_node):
    ids2 = X_node.astype(jnp.int32).reshape(NB, BLK)
    zeros = jnp.zeros((V, D), jnp.float32)
    part = _segment_sum_sc(H, ids2, zeros)
    return _combine(part)


# speedup vs baseline: 7.1181x; 1.0017x over previous
import jax
import jax.numpy as jnp
from jax import lax
from jax.experimental import pallas as pl
from jax.experimental.pallas import tpu as pltpu
from jax.experimental.pallas import tpu_sc as plsc

V = 10000
E = 320000
D = 128

NC = 2
NS = 16
NW = NC * NS
BLK = 128
NB = E // BLK
CHB = 1
CH = CHB * BLK
WBLK = 80
LBLK = NB - (NW - 1) * WBLK
V_TILE = 624
V_LAST = V - (NS - 1) * V_TILE


def _acc_slab(s):
    return pl.ds(s * V_TILE, V_TILE)


def _acc_tail():
    return pl.ds((NS - 1) * V_TILE + V_TILE, V_LAST - V_TILE)


def _sc_body(h_hbm, ids_hbm, zeros_hbm, part_hbm,
             idx_all, rows0, rows1, gsem0, gsem1, ssem0, ssem1, acc):
    c = lax.axis_index("c")
    s = lax.axis_index("s")
    wid = c * NS + s

    @pl.when(wid < NW - 1)
    def _():
        pltpu.sync_copy(ids_hbm.at[pl.ds(wid * WBLK, WBLK)], idx_all)

    @pl.when(wid == NW - 1)
    def _():
        pltpu.sync_copy(ids_hbm.at[pl.ds((NW - 1) * WBLK, LBLK)],
                        idx_all.at[pl.ds(0, LBLK)])

    pltpu.sync_copy(zeros_hbm.at[_acc_slab(s)], acc.at[_acc_slab(s)])

    @pl.when(s == NS - 1)
    def _():
        pltpu.sync_copy(zeros_hbm.at[_acc_tail()], acc.at[_acc_tail()])

    plsc.subcore_barrier()

    b0 = jnp.where(wid < NW - 1, wid * WBLK, (NW - 1) * WBLK)
    nch = jnp.where(wid < NW - 1, WBLK // CHB, LBLK // CHB)
    bufs = ((rows0, gsem0, ssem0), (rows1, gsem1, ssem1))

    def start_gather(ch, k):
        rows, gsem, _ = bufs[k]
        pltpu.async_copy(h_hbm.at[pl.ds((b0 + ch) * BLK, CH)], rows, gsem)

    def wait_gather(k):
        rows, gsem, _ = bufs[k]
        pltpu.make_async_copy(h_hbm.at[pl.ds(0, CH)], rows, gsem).wait()

    def start_scatter(ch, k):
        rows, _, ssem = bufs[k]
        pltpu.async_copy(rows, acc.at[idx_all.at[ch]], ssem, add=True)

    def wait_scatter(k):
        rows, _, ssem = bufs[k]
        pltpu.make_async_copy(rows, acc.at[idx_all.at[0]], ssem).wait()

    start_gather(0, 0)
    start_gather(1, 1)

    def pair(p, carry):
        for k in (0, 1):
            ch = 2 * p + k
            wait_gather(k)
            start_scatter(ch, k)

            @pl.when(ch + 2 < nch)
            def _():
                wait_scatter(k)
                start_gather(ch + 2, k)
        return carry

    lax.fori_loop(0, nch // 2, pair, 0)
    wait_scatter(0)
    wait_scatter(1)
    plsc.subcore_barrier()

    pltpu.sync_copy(acc.at[_acc_slab(s)], part_hbm.at[c, _acc_slab(s)])

    @pl.when(s == NS - 1)
    def _():
        pltpu.sync_copy(acc.at[_acc_tail()], part_hbm.at[c, _acc_tail()])


@jax.jit
def _segment_sum_sc(H, ids2, zeros):
    mesh = plsc.VectorSubcoreMesh(core_axis_name="c", subcore_axis_name="s")
    return pl.kernel(
        _sc_body,
        out_type=jax.ShapeDtypeStruct((NC, V, D), jnp.float32),
        mesh=mesh,
        scratch_types=[
            pltpu.VMEM((WBLK, BLK), jnp.int32),
            pltpu.VMEM((CH, D), jnp.float32),
            pltpu.VMEM((CH, D), jnp.float32),
            pltpu.SemaphoreType.DMA,
            pltpu.SemaphoreType.DMA,
            pltpu.SemaphoreType.DMA,
            pltpu.SemaphoreType.DMA,
            pltpu.VMEM_SHARED((V, D), jnp.float32),
        ],
    )(H, ids2, zeros)


def _tc_add_body(p0, p1, o):
    o[...] = p0[...] + p1[...]


@jax.jit
def _combine(part):
    blk = 1000
    grid = V // blk
    spec = pl.BlockSpec((blk, D), lambda i: (i, 0))
    return pl.pallas_call(
        _tc_add_body,
        grid=(grid,),
        in_specs=[spec, spec],
        out_specs=spec,
        out_shape=jax.ShapeDtypeStruct((V, D), jnp.float32),
    )(part[0], part[1])


def kernel(H, X_node):
    ids2 = X_node.astype(jnp.int32).reshape(NB, BLK)
    zeros = jnp.zeros((V, D), jnp.float32)
    part = _segment_sum_sc(H, ids2, zeros)
    return _combine(part)


# speedup vs baseline: 7.4957x; 1.0530x over previous
import jax
import jax.numpy as jnp
from jax import lax
from jax.experimental import pallas as pl
from jax.experimental.pallas import tpu as pltpu
from jax.experimental.pallas import tpu_sc as plsc

V = 10000
E = 320000
D = 128

NC = 2
NS = 16
NW = NC * NS
BLK = 128
NB = E // BLK
NBUF = 3
WBLK = NB // NW
NTRI = WBLK // NBUF
TAIL = NB - NW * WBLK
V_TILE = 624
V_LAST = V - (NS - 1) * V_TILE


def _acc_slab(s):
    return pl.ds(s * V_TILE, V_TILE)


def _acc_tail():
    return pl.ds((NS - 1) * V_TILE + V_TILE, V_LAST - V_TILE)


def _sc_body(h_hbm, ids_hbm, zeros_hbm, part_hbm, *scratch):
    rows = scratch[0:NBUF]
    idxs = scratch[NBUF:2 * NBUF]
    gsems = scratch[2 * NBUF:3 * NBUF]
    ssems = scratch[3 * NBUF:4 * NBUF]
    acc = scratch[4 * NBUF]

    c = lax.axis_index("c")
    s = lax.axis_index("s")
    wid = c * NS + s

    pltpu.sync_copy(zeros_hbm.at[_acc_slab(s)], acc.at[_acc_slab(s)])

    @pl.when(s == NS - 1)
    def _():
        pltpu.sync_copy(zeros_hbm.at[_acc_tail()], acc.at[_acc_tail()])

    plsc.subcore_barrier()

    b0 = wid * WBLK

    def start_gather(ch, k):
        pltpu.async_copy(h_hbm.at[pl.ds((b0 + ch) * BLK, BLK)], rows[k],
                         gsems[k])
        pltpu.async_copy(ids_hbm.at[pl.ds((b0 + ch) * BLK, BLK)], idxs[k],
                         gsems[k])

    def wait_gather(k):
        pltpu.make_async_copy(h_hbm.at[pl.ds(0, BLK)], rows[k],
                              gsems[k]).wait()
        pltpu.make_async_copy(ids_hbm.at[pl.ds(0, BLK)], idxs[k],
                              gsems[k]).wait()

    def start_scatter(k):
        pltpu.async_copy(rows[k], acc.at[idxs[k]], ssems[k], add=True)

    def wait_scatter(k):
        pltpu.make_async_copy(rows[k], acc.at[idxs[k]], ssems[k]).wait()

    for k in range(NBUF):
        start_gather(k, k)

    def triple(p, carry):
        for k in range(NBUF):
            ch = NBUF * p + k
            wait_gather(k)
            start_scatter(k)

            @pl.when(ch + NBUF < WBLK)
            def _():
                wait_scatter(k)
                start_gather(ch + NBUF, k)
        return carry

    lax.fori_loop(0, NTRI, triple, 0)
    for k in range(NBUF):
        wait_scatter(k)

    @pl.when(wid < TAIL)
    def _():
        tb = NW * WBLK + wid
        pltpu.sync_copy(h_hbm.at[pl.ds(tb * BLK, BLK)], rows[0])
        pltpu.sync_copy(ids_hbm.at[pl.ds(tb * BLK, BLK)], idxs[0])
        pltpu.sync_copy(rows[0], acc.at[idxs[0]], add=True)

    plsc.subcore_barrier()

    pltpu.sync_copy(acc.at[_acc_slab(s)], part_hbm.at[c, _acc_slab(s)])

    @pl.when(s == NS - 1)
    def _():
        pltpu.sync_copy(acc.at[_acc_tail()], part_hbm.at[c, _acc_tail()])


@jax.jit
def _segment_sum_sc(H, ids, zeros):
    mesh = plsc.VectorSubcoreMesh(core_axis_name="c", subcore_axis_name="s")
    return pl.kernel(
        _sc_body,
        out_type=jax.ShapeDtypeStruct((NC, V, D), jnp.float32),
        mesh=mesh,
        scratch_types=(
            [pltpu.VMEM((BLK, D), jnp.float32) for _ in range(NBUF)]
            + [pltpu.VMEM((BLK,), jnp.int32) for _ in range(NBUF)]
            + [pltpu.SemaphoreType.DMA for _ in range(2 * NBUF)]
            + [pltpu.VMEM_SHARED((V, D), jnp.float32)]
        ),
    )(H, ids, zeros)


def _tc_add_body(p0, p1, o):
    o[...] = p0[...] + p1[...]


@jax.jit
def _combine(part):
    blk = 1000
    grid = V // blk
    spec = pl.BlockSpec((blk, D), lambda i: (i, 0))
    return pl.pallas_call(
        _tc_add_body,
        grid=(grid,),
        in_specs=[spec, spec],
        out_specs=spec,
        out_shape=jax.ShapeDtypeStruct((V, D), jnp.float32),
    )(part[0], part[1])


def kernel(H, X_node):
    ids = X_node.astype(jnp.int32)
    zeros = jnp.zeros((V, D), jnp.float32)
    part = _segment_sum_sc(H, ids, zeros)
    return _combine(part)


# speedup vs baseline: 7.7127x; 1.0290x over previous
import jax
import jax.numpy as jnp
from jax import lax
from jax.experimental import pallas as pl
from jax.experimental.pallas import tpu as pltpu
from jax.experimental.pallas import tpu_sc as plsc

V = 10000
E = 320000
D = 128

NC = 2
NS = 16
NW = NC * NS
BLK = 128
NB = E // BLK
NBUF = 3
WBLK = NB // NW
NTRI = WBLK // NBUF
TAIL = NB - NW * WBLK
V_TILE = 624
V_LAST = V - (NS - 1) * V_TILE


def _acc_slab(s):
    return pl.ds(s * V_TILE, V_TILE)


def _acc_tail():
    return pl.ds((NS - 1) * V_TILE + V_TILE, V_LAST - V_TILE)


def _sc_body(h_hbm, ids_hbm, part_hbm, *scratch):
    rows = scratch[0:NBUF]
    idxs = scratch[NBUF:2 * NBUF]
    gsems = scratch[2 * NBUF:3 * NBUF]
    ssems = scratch[3 * NBUF:4 * NBUF]
    acc = scratch[4 * NBUF]

    c = lax.axis_index("c")
    s = lax.axis_index("s")
    wid = c * NS + s

    z16 = jnp.zeros((16,), jnp.float32)

    def zrow(r, carry):
        for g in range(D // 16):
            rows[0][r, pl.ds(g * 16, 16)] = z16
        return carry

    lax.fori_loop(0, BLK, zrow, 0)
    for r0 in range(0, V_TILE, BLK):
        n = min(BLK, V_TILE - r0)
        pltpu.sync_copy(rows[0].at[pl.ds(0, n)],
                        acc.at[pl.ds(s * V_TILE + r0, n)])

    @pl.when(s == NS - 1)
    def _():
        pltpu.sync_copy(rows[0].at[pl.ds(0, V_LAST - V_TILE)],
                        acc.at[_acc_tail()])

    plsc.subcore_barrier()

    b0 = wid * WBLK

    def start_gather(ch, k):
        pltpu.async_copy(h_hbm.at[pl.ds((b0 + ch) * BLK, BLK)], rows[k],
                         gsems[k])
        pltpu.async_copy(ids_hbm.at[pl.ds((b0 + ch) * BLK, BLK)], idxs[k],
                         gsems[k])

    def wait_gather(k):
        pltpu.make_async_copy(h_hbm.at[pl.ds(0, BLK)], rows[k],
                              gsems[k]).wait()
        pltpu.make_async_copy(ids_hbm.at[pl.ds(0, BLK)], idxs[k],
                              gsems[k]).wait()

    def start_scatter(k):
        pltpu.async_copy(rows[k], acc.at[idxs[k]], ssems[k], add=True)

    def wait_scatter(k):
        pltpu.make_async_copy(rows[k], acc.at[idxs[k]], ssems[k]).wait()

    for k in range(NBUF):
        start_gather(k, k)

    def triple(p, carry):
        for k in range(NBUF):
            ch = NBUF * p + k
            wait_gather(k)
            start_scatter(k)

            @pl.when(ch + NBUF < WBLK)
            def _():
                wait_scatter(k)
                start_gather(ch + NBUF, k)
        return carry

    lax.fori_loop(0, NTRI, triple, 0)
    for k in range(NBUF):
        wait_scatter(k)

    @pl.when(wid < TAIL)
    def _():
        tb = NW * WBLK + wid
        pltpu.sync_copy(h_hbm.at[pl.ds(tb * BLK, BLK)], rows[0])
        pltpu.sync_copy(ids_hbm.at[pl.ds(tb * BLK, BLK)], idxs[0])
        pltpu.sync_copy(rows[0], acc.at[idxs[0]], add=True)

    plsc.subcore_barrier()

    pltpu.sync_copy(acc.at[_acc_slab(s)], part_hbm.at[c, _acc_slab(s)])

    @pl.when(s == NS - 1)
    def _():
        pltpu.sync_copy(acc.at[_acc_tail()], part_hbm.at[c, _acc_tail()])


@jax.jit
def _segment_sum_sc(H, ids):
    mesh = plsc.VectorSubcoreMesh(core_axis_name="c", subcore_axis_name="s")
    return pl.kernel(
        _sc_body,
        out_type=jax.ShapeDtypeStruct((NC, V, D), jnp.float32),
        mesh=mesh,
        scratch_types=(
            [pltpu.VMEM((BLK, D), jnp.float32) for _ in range(NBUF)]
            + [pltpu.VMEM((BLK,), jnp.int32) for _ in range(NBUF)]
            + [pltpu.SemaphoreType.DMA for _ in range(2 * NBUF)]
            + [pltpu.VMEM_SHARED((V, D), jnp.float32)]
        ),
    )(H, ids)


def _tc_add_body(p0, p1, o):
    o[...] = p0[...] + p1[...]


@jax.jit
def _combine(part):
    blk = 1000
    grid = V // blk
    spec = pl.BlockSpec((blk, D), lambda i: (i, 0))
    return pl.pallas_call(
        _tc_add_body,
        grid=(grid,),
        in_specs=[spec, spec],
        out_specs=spec,
        out_shape=jax.ShapeDtypeStruct((V, D), jnp.float32),
    )(part[0], part[1])


def kernel(H, X_node):
    ids = X_node.astype(jnp.int32)
    part = _segment_sum_sc(H, ids)
    return _combine(part)
